# baseline (speedup 1.0000x reference)
import functools

import jax
import jax.numpy as jnp
from jax import lax
from jax.experimental import pallas as pl
from jax.experimental.pallas import tpu as pltpu
from jax.experimental.pallas import tpu_sc as plsc

SELECTION_RATIO = 0.2

_NUM_CORES = 2
_NUM_SUBCORES = 16
_PER_W = 16


def _topk_idx_kernel(att_ref, att_t_ref, idx_ref, *, n, p_pad, n_chunk):
    s_row = jnp.mean(att_ref[...], axis=0, keepdims=True)
    s_col = jnp.mean(att_t_ref[...], axis=1, keepdims=True)
    ii = lax.broadcasted_iota(jnp.int32, (n, n_chunk), 0)
    cnt = jnp.zeros((n, 1), jnp.int32)
    for c in range(n // n_chunk):
        sr = lax.slice(s_row, (0, c * n_chunk), (1, (c + 1) * n_chunk))
        jj = lax.broadcasted_iota(jnp.int32, (n, n_chunk), 1) + c * n_chunk
        gt = sr > s_col
        eq = (sr == s_col) & (jj < ii)
        cnt = cnt + jnp.sum((gt | eq).astype(jnp.int32), axis=1, keepdims=True)
    pp = lax.broadcasted_iota(jnp.int32, (n, p_pad), 1)
    i_col = lax.broadcasted_iota(jnp.int32, (n, p_pad), 0)
    contrib = jnp.where(cnt == pp, i_col, 0)
    idx_ref[...] = jnp.sum(contrib, axis=0, keepdims=True)


def _make_gather(l_layers, s_tokens, d, n_active, per_w):
    mesh = plsc.VectorSubcoreMesh(core_axis_name="c", subcore_axis_name="s")
    kpad = n_active * per_w
    out_sds = jax.ShapeDtypeStruct((l_layers, kpad, d), jnp.float32)

    @functools.partial(
        pl.kernel,
        mesh=mesh,
        out_type=[out_sds, out_sds],
        scratch_types=[
            pltpu.VMEM((per_w,), jnp.int32),
            pltpu.VMEM((per_w,), jnp.int32),
            pltpu.VMEM((per_w, d), jnp.float32),
            pltpu.VMEM((per_w, d), jnp.float32),
            pltpu.SemaphoreType.DMA,
            pltpu.SemaphoreType.DMA,
        ],
    )
    def gather(keys_hbm, vals_hbm, idx_hbm, kout_hbm, vout_hbm,
               idx_v, gidx_v, krows_v, vrows_v, semk, semv):
        wid = lax.axis_index("s") * _NUM_CORES + lax.axis_index("c")

        @pl.when(wid < n_active)
        def _():
            base = wid * per_w
            pltpu.sync_copy(idx_hbm.at[pl.ds(base, per_w)], idx_v)
            for l in range(l_layers):
                gidx_v[...] = idx_v[...] + (l * s_tokens + 1)
                ck = pltpu.async_copy(keys_hbm.at[gidx_v], krows_v, semk)
                cv = pltpu.async_copy(vals_hbm.at[gidx_v], vrows_v, semv)
                ck.wait()
                pltpu.sync_copy(krows_v, kout_hbm.at[l, pl.ds(base, per_w)])
                cv.wait()
                pltpu.sync_copy(vrows_v, vout_hbm.at[l, pl.ds(base, per_w)])

    return gather


def _fuse_head_kernel(kx_ref, vx_ref, wf_ref, xc_ref, wh_ref, out_ref,
                      acc_ref, *, k_sel, denom, nsteps):
    l = pl.program_id(0)
    x = kx_ref[0] + vx_ref[0]
    t = jnp.tanh(jnp.dot(x, wf_ref[...], preferred_element_type=jnp.float32))
    mask = lax.broadcasted_iota(jnp.int32, t.shape, 0) < k_sel
    part = jnp.sum(jnp.where(mask, t, 0.0), axis=0, keepdims=True)

    @pl.when(l == 0)
    def _():
        acc_ref[...] = part

    @pl.when(l > 0)
    def _():
        acc_ref[...] = acc_ref[...] + part

    @pl.when(l == nsteps - 1)
    def _():
        fine = acc_ref[...] * (1.0 / denom)
        xf = xc_ref[...] + fine
        out_ref[...] = jnp.dot(xf, wh_ref[...],
                               preferred_element_type=jnp.float32)


def kernel(x_coarse, key_states, value_states, attention_map, W_fuse, W_head):
    l_layers, b, s_tokens, d = key_states.shape
    h = attention_map.shape[1]
    n = s_tokens - 1
    k_sel = max(1, int(n * SELECTION_RATIO))
    num_classes = W_head.shape[1]

    n_active = -(-k_sel // _PER_W)
    kpad = n_active * _PER_W
    p_pad = -(-kpad // 128) * 128

    att = attention_map[0, :, 0, 1:]
    att_t = att.T
    idx = pl.pallas_call(
        functools.partial(_topk_idx_kernel, n=n, p_pad=p_pad, n_chunk=128),
        out_shape=jax.ShapeDtypeStruct((1, p_pad), jnp.int32),
    )(att, att_t)
    idx_flat = idx.reshape(p_pad)

    keys2d = key_states.reshape(l_layers * s_tokens, d)
    vals2d = value_states.reshape(l_layers * s_tokens, d)
    gather = _make_gather(l_layers, s_tokens, d, n_active, _PER_W)
    kx, vx = gather(keys2d, vals2d, idx_flat)

    logits = pl.pallas_call(
        functools.partial(_fuse_head_kernel, k_sel=k_sel,
                          denom=float(l_layers * k_sel), nsteps=l_layers),
        grid=(l_layers,),
        in_specs=[
            pl.BlockSpec((1, kpad, d), lambda l: (l, 0, 0)),
            pl.BlockSpec((1, kpad, d), lambda l: (l, 0, 0)),
            pl.BlockSpec((d, d), lambda l: (0, 0)),
            pl.BlockSpec((b, d), lambda l: (0, 0)),
            pl.BlockSpec((d, num_classes), lambda l: (0, 0)),
        ],
        out_specs=pl.BlockSpec((b, num_classes), lambda l: (0, 0)),
        scratch_shapes=[pltpu.VMEM((1, d), jnp.float32)],
        out_shape=jax.ShapeDtypeStruct((b, num_classes), jnp.float32),
    )(kx, vx, W_fuse, x_coarse, W_head)
    return logits
